# baseline (speedup 1.0000x reference)
import jax
import jax.numpy as jnp
from jax import lax
from jax.experimental import pallas as pl
from jax.experimental.pallas import tpu as pltpu

_TN = 256


def _half(xh, b1s, bias1p, b2s, bias2p, ss, w1s, bfc1_ref, w2b, bfc2_ref, tn):
    rq = tn * 7
    r1 = rq - 1
    r2 = rq - 3
    bf = jnp.bfloat16

    xb = jnp.pad(xh.astype(bf), ((0, 0), (0, 16)))
    xw = jnp.concatenate([xb[0:r1], xb[1:r1 + 1]], axis=1)
    y1 = jnp.dot(xw, b1s[...], preferred_element_type=jnp.float32)

    p1 = jnp.maximum(y1[:, 0:512], y1[:, 512:1024])
    p2 = jnp.maximum(y1[:, 1024:1536], y1[:, 1536:2048])
    x1e = jnp.maximum(jnp.maximum(p1[:, 0:256], p1[:, 256:512]).astype(bf)
                      + bias1p, 0)
    x1o = jnp.maximum(jnp.maximum(p2[:, 0:256], p2[:, 256:512]).astype(bf)
                      + bias1p, 0)

    xw3 = jnp.concatenate(
        [x1e[0:r2], x1o[0:r2], x1e[1:r2 + 1], x1o[1:r2 + 1],
         x1e[2:r2 + 2], x1o[2:r2 + 2]], axis=1)
    y2a = jnp.dot(xw3[:, 0:1280], b2s[...],
                  preferred_element_type=jnp.float32)
    y2b = jnp.dot(xw3[:, 256:1536], b2s[...],
                  preferred_element_type=jnp.float32)

    v2 = jnp.maximum(y2a, y2b)
    x2 = jnp.maximum(jnp.maximum(v2[:, 0:256], v2[:, 256:512]).astype(bf)
                     + bias2p, 0)

    a = jnp.dot(ss[...], x2,
                preferred_element_type=jnp.float32).astype(bf)
    af = jnp.concatenate([a[i3 * tn:(i3 + 1) * tn] for i3 in range(4)],
                         axis=1)
    h = jnp.dot(af, w1s[...], preferred_element_type=jnp.float32)
    h = jnp.maximum(h + bfc1_ref[...], 0.0).astype(bf)
    logits = jnp.dot(h, w2b, preferred_element_type=jnp.float32)
    return logits + bfc2_ref[...]


def _fused_body(x_ref, b1w_ref, bias1_ref, b2w_ref, bias2_ref, wfc1_ref,
                bfc1_ref, wfc2_ref, bfc2_ref, o_ref, b1s, b2s, ss, w1s):
    nsub = 4
    tn = _TN // nsub
    rq = tn * 7
    r2 = rq - 3
    bf = jnp.bfloat16

    @pl.when(pl.program_id(0) == 0)
    def _prep():
        b1s[...] = jnp.zeros((256, 2048), bf)
        for r in range(4):
            for ki in range(5):
                u, rr = divmod(r + ki, 4)
                rb = u * 128 + rr * 28
                b1s[rb:rb + 28, r * 512:(r + 1) * 512] = \
                    b1w_ref[ki].astype(bf)
        b2s[...] = jnp.zeros((1280, 512), bf)
        for ki in range(5):
            b2s[ki * 256:ki * 256 + 240, :] = b2w_ref[ki].astype(bf)
        gi = lax.broadcasted_iota(jnp.int32, (4 * tn, r2), 0)
        ci = lax.broadcasted_iota(jnp.int32, (4 * tn, r2), 1)
        col = 7 * lax.rem(gi, tn) + lax.div(gi, tn)
        ss[...] = jnp.where(ci == col, 1.0, 0.0).astype(bf)
        w1s[...] = jnp.zeros((1024, 500), bf)
        for i3 in range(4):
            w1s[i3 * 256:i3 * 256 + 200, :] = wfc1_ref[i3].astype(bf)

    bias1p = jnp.pad(bias1_ref[...].astype(bf), ((0, 0), (0, 16)))
    bias2p = jnp.pad(bias2_ref[...].astype(bf), ((0, 0), (0, 56)))
    w2b = wfc2_ref[...].astype(bf)

    args = (b1s, bias1p, b2s, bias2p, ss, w1s, bfc1_ref, w2b, bfc2_ref, tn)
    for q in range(nsub):
        o_ref[q * tn:(q + 1) * tn, :] = _half(x_ref[q * rq:(q + 1) * rq, :], *args)


def kernel(b1w, bias1, b2w, bias2, wfc1, bfc1, wfc2, bfc2, x):
    n = x.shape[0]
    tn = _TN
    npad = ((n + tn - 1) // tn) * tn
    xr = x.reshape(n, 28, 28)
    if npad != n:
        xr = jnp.concatenate(
            [xr, jnp.zeros((npad - n, 28, 28), x.dtype)], axis=0)
    xq = xr.reshape(npad * 7, 112)

    steps = npad // tn
    rq = tn * 7
    qn = tn // 4
    bf = jnp.bfloat16
    out = pl.pallas_call(
        _fused_body,
        out_shape=jax.ShapeDtypeStruct((npad, 10), jnp.float32),
        grid=(steps,),
        in_specs=[
            pl.BlockSpec((rq, 112), lambda i: (i, 0)),
            pl.BlockSpec((5, 28, 512), lambda i: (0, 0, 0)),
            pl.BlockSpec((1, 240), lambda i: (0, 0)),
            pl.BlockSpec((5, 240, 512), lambda i: (0, 0, 0)),
            pl.BlockSpec((1, 200), lambda i: (0, 0)),
            pl.BlockSpec((4, 200, 500), lambda i: (0, 0, 0)),
            pl.BlockSpec((1, 500), lambda i: (0, 0)),
            pl.BlockSpec((500, 10), lambda i: (0, 0)),
            pl.BlockSpec((1, 10), lambda i: (0, 0)),
        ],
        out_specs=pl.BlockSpec((tn, 10), lambda i: (i, 0)),
        scratch_shapes=[
            pltpu.VMEM((256, 2048), bf),
            pltpu.VMEM((1280, 512), bf),
            pltpu.VMEM((4 * qn, qn * 7 - 3), bf),
            pltpu.VMEM((1024, 500), bf),
        ],
        compiler_params=pltpu.CompilerParams(
            dimension_semantics=("arbitrary",),
            vmem_limit_bytes=64 * 1024 * 1024),
    )(xq, b1w, bias1, b2w, bias2, wfc1, bfc1, wfc2, bfc2)

    return out[:n] if npad != n else out


# speedup vs baseline: 1.0394x; 1.0394x over previous
import jax
import jax.numpy as jnp
from jax import lax
from jax.experimental import pallas as pl
from jax.experimental.pallas import tpu as pltpu

_TN = 256


def _half(xh, b1s, bias1p, b2s, bias2p, ss, w1s, bfc1_ref, w2b, bfc2_ref, tn):
    rq = tn * 7
    r1 = rq - 1
    r2 = rq - 3
    bf = jnp.bfloat16

    xb = jnp.pad(xh.astype(bf), ((0, 0), (0, 16)))
    xw = jnp.concatenate([xb[0:r1], xb[1:r1 + 1]], axis=1)
    y1 = jnp.dot(xw, b1s[...], preferred_element_type=jnp.float32)

    p1 = jnp.maximum(y1[:, 0:512], y1[:, 512:1024])
    p2 = jnp.maximum(y1[:, 1024:1536], y1[:, 1536:2048])
    x1e = jnp.maximum(jnp.maximum(p1[:, 0:256], p1[:, 256:512]).astype(bf)
                      + bias1p, 0)
    x1o = jnp.maximum(jnp.maximum(p2[:, 0:256], p2[:, 256:512]).astype(bf)
                      + bias1p, 0)

    xw3 = jnp.concatenate(
        [x1e[0:r2], x1o[0:r2], x1e[1:r2 + 1], x1o[1:r2 + 1],
         x1e[2:r2 + 2], x1o[2:r2 + 2]], axis=1)
    y2a = jnp.dot(xw3[:, 0:1280], b2s[...],
                  preferred_element_type=jnp.float32)
    y2b = jnp.dot(xw3[:, 256:1536], b2s[...],
                  preferred_element_type=jnp.float32)

    v2 = jnp.maximum(y2a, y2b)
    x2 = jnp.maximum(jnp.maximum(v2[:, 0:256], v2[:, 256:512]).astype(bf)
                     + bias2p, 0)

    a = jnp.dot(ss[...], x2,
                preferred_element_type=jnp.float32).astype(bf)
    af = jnp.concatenate([a[i3 * tn:(i3 + 1) * tn] for i3 in range(4)],
                         axis=1)
    h = jnp.dot(af, w1s[...], preferred_element_type=jnp.float32)
    h = jnp.maximum(h + bfc1_ref[...], 0.0).astype(bf)
    logits = jnp.dot(h, w2b, preferred_element_type=jnp.float32)
    return logits + bfc2_ref[...]


def _fused_body(x_ref, b1w_ref, bias1_ref, b2w_ref, bias2_ref, wfc1_ref,
                bfc1_ref, wfc2_ref, bfc2_ref, o_ref, b1s, b2s, ss, w1s):
    tn = _TN // 2
    rq = tn * 7
    r2 = rq - 3
    bf = jnp.bfloat16

    @pl.when(pl.program_id(0) == 0)
    def _prep():
        b1s[...] = jnp.zeros((256, 2048), bf)
        for r in range(4):
            for ki in range(5):
                u, rr = divmod(r + ki, 4)
                rb = u * 128 + rr * 28
                b1s[rb:rb + 28, r * 512:(r + 1) * 512] = \
                    b1w_ref[ki].astype(bf)
        b2s[...] = jnp.zeros((1280, 512), bf)
        for ki in range(5):
            b2s[ki * 256:ki * 256 + 240, :] = b2w_ref[ki].astype(bf)
        gi = lax.broadcasted_iota(jnp.int32, (4 * tn, r2), 0)
        ci = lax.broadcasted_iota(jnp.int32, (4 * tn, r2), 1)
        col = 7 * lax.rem(gi, tn) + lax.div(gi, tn)
        ss[...] = jnp.where(ci == col, 1.0, 0.0).astype(bf)
        w1s[...] = jnp.zeros((1024, 500), bf)
        for i3 in range(4):
            w1s[i3 * 256:i3 * 256 + 200, :] = wfc1_ref[i3].astype(bf)

    bias1p = jnp.pad(bias1_ref[...].astype(bf), ((0, 0), (0, 16)))
    bias2p = jnp.pad(bias2_ref[...].astype(bf), ((0, 0), (0, 56)))
    w2b = wfc2_ref[...].astype(bf)

    args = (b1s, bias1p, b2s, bias2p, ss, w1s, bfc1_ref, w2b, bfc2_ref, tn)
    o_ref[0:tn, :] = _half(x_ref[0:rq, :], *args)
    o_ref[tn:2 * tn, :] = _half(x_ref[rq:2 * rq, :], *args)


def kernel(b1w, bias1, b2w, bias2, wfc1, bfc1, wfc2, bfc2, x):
    n = x.shape[0]
    tn = _TN
    npad = ((n + tn - 1) // tn) * tn
    xr = x.reshape(n, 28, 28)
    if npad != n:
        xr = jnp.concatenate(
            [xr, jnp.zeros((npad - n, 28, 28), x.dtype)], axis=0)
    xq = xr.reshape(npad * 7, 112)

    steps = npad // tn
    rq = tn * 7
    hn = tn // 2
    bf = jnp.bfloat16
    out = pl.pallas_call(
        _fused_body,
        out_shape=jax.ShapeDtypeStruct((npad, 10), jnp.float32),
        grid=(steps,),
        in_specs=[
            pl.BlockSpec((rq, 112), lambda i: (i, 0)),
            pl.BlockSpec((5, 28, 512), lambda i: (0, 0, 0)),
            pl.BlockSpec((1, 240), lambda i: (0, 0)),
            pl.BlockSpec((5, 240, 512), lambda i: (0, 0, 0)),
            pl.BlockSpec((1, 200), lambda i: (0, 0)),
            pl.BlockSpec((4, 200, 500), lambda i: (0, 0, 0)),
            pl.BlockSpec((1, 500), lambda i: (0, 0)),
            pl.BlockSpec((500, 10), lambda i: (0, 0)),
            pl.BlockSpec((1, 10), lambda i: (0, 0)),
        ],
        out_specs=pl.BlockSpec((tn, 10), lambda i: (i, 0)),
        scratch_shapes=[
            pltpu.VMEM((256, 2048), bf),
            pltpu.VMEM((1280, 512), bf),
            pltpu.VMEM((4 * hn, hn * 7 - 3), bf),
            pltpu.VMEM((1024, 500), bf),
        ],
        compiler_params=pltpu.CompilerParams(
            dimension_semantics=("arbitrary",),
            vmem_limit_bytes=64 * 1024 * 1024),
    )(xq, b1w, bias1, b2w, bias2, wfc1, bfc1, wfc2, bfc2)

    return out[:n] if npad != n else out
